# baseline (speedup 1.0000x reference)
import jax
import jax.numpy as jnp
from jax import lax
from jax.experimental import pallas as pl
from jax.experimental.pallas import tpu as pltpu
from jax.experimental.pallas import tpu_sc as plsc

N_SITES = 20
N_STATES = 2 ** N_SITES
BATCH = 16384

NUM_CORES = 2
NUM_SUBCORES = 16
LANES = 16
NUM_WORKERS = NUM_CORES * NUM_SUBCORES
B_PER_W = BATCH // NUM_WORKERS
CHUNK = 128
N_CHUNKS = B_PER_W // CHUNK
GROUPS_PER_CHUNK = CHUNK // LANES


def _sc_body(xt_hbm, table_hbm, out_hbm, x_v, idx_v, out_v, gsem):
    wid = lax.axis_index("s") * NUM_CORES + lax.axis_index("c")
    base = wid * B_PER_W

    pltpu.sync_copy(xt_hbm.at[:, pl.ds(base, B_PER_W)], x_v)

    gathers = []
    for j in range(N_CHUNKS):
        for gg in range(GROUPS_PER_CHUNK):
            sl = pl.ds((j * GROUPS_PER_CHUNK + gg) * LANES, LANES)
            num = x_v[0, sl]
            for s in range(1, N_SITES):
                num = num * 2 + x_v[s, sl]
            idx_v[sl] = num
        cs = pl.ds(j * CHUNK, CHUNK)
        gathers.append(
            pltpu.async_copy(table_hbm.at[idx_v.at[cs]], out_v.at[cs], gsem))
    for c in gathers:
        c.wait()

    pltpu.sync_copy(out_v, out_hbm.at[pl.ds(base, B_PER_W)])


@jax.jit
def kernel(x_in, logstate):
    xt = x_in.T
    mesh = plsc.VectorSubcoreMesh(core_axis_name="c", subcore_axis_name="s")
    run = pl.kernel(
        _sc_body,
        mesh=mesh,
        out_type=jax.ShapeDtypeStruct((BATCH,), jnp.float32),
        scratch_types=[
            pltpu.VMEM((N_SITES, B_PER_W), jnp.int32),
            pltpu.VMEM((B_PER_W,), jnp.int32),
            pltpu.VMEM((B_PER_W,), jnp.float32),
            pltpu.SemaphoreType.DMA,
        ],
    )
    return run(xt, logstate)


# speedup vs baseline: 1.0495x; 1.0495x over previous
import jax
import jax.numpy as jnp
from jax import lax
from jax.experimental import pallas as pl
from jax.experimental.pallas import tpu as pltpu
from jax.experimental.pallas import tpu_sc as plsc

N_SITES = 20
N_STATES = 2 ** N_SITES
BATCH = 16384

NUM_CORES = 2
NUM_SUBCORES = 16
LANES = 16
NUM_WORKERS = NUM_CORES * NUM_SUBCORES
B_PER_W = BATCH // NUM_WORKERS
CHUNK = 128
N_CHUNKS = B_PER_W // CHUNK
GROUPS_PER_CHUNK = CHUNK // LANES


def _sc_body(xt_hbm, table_hbm, out_hbm, x_v, idx_v, out_v, gsem):
    wid = lax.axis_index("s") * NUM_CORES + lax.axis_index("c")
    base = wid * B_PER_W

    pltpu.sync_copy(xt_hbm.at[:, pl.ds(base, B_PER_W)], x_v)

    def pack_group(g, _):
        sl = pl.ds(g * LANES, LANES)
        num = x_v[0, sl]
        for s in range(1, N_SITES):
            num = num * 2 + x_v[s, sl]
        idx_v[sl] = num
        return _

    lax.fori_loop(0, B_PER_W // LANES, pack_group, None)

    gathers = []
    for j in range(N_CHUNKS):
        cs = pl.ds(j * CHUNK, CHUNK)
        gathers.append(
            pltpu.async_copy(table_hbm.at[idx_v.at[cs]], out_v.at[cs], gsem))
    for c in gathers:
        c.wait()

    pltpu.sync_copy(out_v, out_hbm.at[pl.ds(base, B_PER_W)])


@jax.jit
def kernel(x_in, logstate):
    xt = x_in.T
    mesh = plsc.VectorSubcoreMesh(core_axis_name="c", subcore_axis_name="s")
    run = pl.kernel(
        _sc_body,
        mesh=mesh,
        out_type=jax.ShapeDtypeStruct((BATCH,), jnp.float32),
        scratch_types=[
            pltpu.VMEM((N_SITES, B_PER_W), jnp.int32),
            pltpu.VMEM((B_PER_W,), jnp.int32),
            pltpu.VMEM((B_PER_W,), jnp.float32),
            pltpu.SemaphoreType.DMA,
        ],
    )
    return run(xt, logstate)
